# baseline (speedup 1.0000x reference)
import functools

import jax
import jax.numpy as jnp
from jax import lax
from jax.experimental import pallas as pl
from jax.experimental.pallas import tpu as pltpu
from jax.experimental.pallas import tpu_sc as plsc

_L = 16
_NTILES = 16
_NCORES = 2
_CHUNK = 9216
_BATCH = 128
_ZR = 32
_W = 144


def _ceil_to(x, m):
    return (x + m - 1) // m * m


def _augment(x):
    n = x.shape[0]
    return jnp.concatenate(
        [x, jnp.ones((n, 1), jnp.float32),
         jnp.zeros((n, _W - 129), jnp.float32)], axis=1)


def _seg_aggregate(x_aug, src, dst, n_dst):
    e = src.shape[0]
    e_pad = _ceil_to(e, _NTILES * _BATCH)
    if e_pad != e:
        pad = e_pad - e
        src = jnp.concatenate([src, jnp.zeros((pad,), jnp.int32)])
        dst = jnp.concatenate([dst, jnp.full((pad,), jnp.int32(1 << 30))])
    epw = e_pad // _NTILES
    n_batches = epw // _BATCH
    n_chunks = -(-n_dst // _CHUNK)
    n_pad = n_chunks * _CHUNK
    rpt = _CHUNK // _NTILES

    mesh = plsc.VectorSubcoreMesh(core_axis_name="c", subcore_axis_name="s")

    @functools.partial(
        pl.kernel,
        out_type=(
            jax.ShapeDtypeStruct((n_pad, 128), jnp.float32),
            jax.ShapeDtypeStruct((n_pad, _L), jnp.float32),
        ),
        mesh=mesh,
        compiler_params=pltpu.CompilerParams(use_tc_tiling_on_sc=False),
        scratch_types=dict(
            esrc=pltpu.VMEM((_BATCH,), jnp.int32),
            edst=pltpu.VMEM((_BATCH,), jnp.int32),
            rows=pltpu.VMEM((_BATCH, _W), jnp.float32),
            gidx=pltpu.VMEM((_BATCH,), jnp.int32),
            sidx=pltpu.VMEM((_BATCH,), jnp.int32),
            zbuf=pltpu.VMEM((_ZR, _W), jnp.float32),
            acc_sh=pltpu.VMEM_SHARED((_CHUNK + _L, _W), jnp.float32),
        ),
    )
    def k(x_hbm, src_hbm, dst_hbm, agg_hbm, deg_hbm, *,
          esrc, edst, rows, gidx, sidx, zbuf, acc_sh):
        cid = lax.axis_index("c")
        sid = lax.axis_index("s")
        ebase = sid * epw

        for i in range(_ZR):
            for j in range(_W // _L):
                zbuf[i, pl.ds(j * _L, _L)] = jnp.zeros((_L,), jnp.float32)

        def do_batch(lo, b):
            estart = ebase + b * _BATCH
            pltpu.sync_copy(src_hbm.at[pl.ds(estart, _BATCH)], esrc)
            pltpu.sync_copy(dst_hbm.at[pl.ds(estart, _BATCH)], edst)
            for j in range(_BATCH // _L):
                s_v = esrc[pl.ds(j * _L, _L)]
                d_v = edst[pl.ds(j * _L, _L)]
                m = (d_v >= lo) & (d_v < lo + _CHUNK)
                gidx[pl.ds(j * _L, _L)] = jnp.where(m, s_v, 0)
                sidx[pl.ds(j * _L, _L)] = jnp.where(m, d_v - lo, _CHUNK)
            pltpu.sync_copy(x_hbm.at[gidx], rows)
            pltpu.sync_copy(rows, acc_sh.at[sidx], add=True)

        rbase = sid * rpt
        def pair_body(i, carry):
            my_chunk = i * _NCORES + cid
            active = my_chunk < n_chunks
            lo = my_chunk * _CHUNK

            @pl.when(active)
            def _zero():
                def zloop(z, c2):
                    pltpu.sync_copy(
                        zbuf, acc_sh.at[pl.ds(rbase + z * _ZR, _ZR), :])
                    return c2
                lax.fori_loop(0, rpt // _ZR, zloop, 0)
            plsc.subcore_barrier()

            @pl.when(active)
            def _scan():
                def bloop(b, c2):
                    do_batch(lo, b)
                    return c2
                lax.fori_loop(0, n_batches, bloop, 0)
            plsc.subcore_barrier()

            @pl.when(active)
            def _writeout():
                pltpu.sync_copy(
                    acc_sh.at[pl.ds(rbase, rpt), pl.ds(0, 128)],
                    agg_hbm.at[pl.ds(lo + rbase, rpt), :])
                pltpu.sync_copy(
                    acc_sh.at[pl.ds(rbase, rpt), pl.ds(128, _L)],
                    deg_hbm.at[pl.ds(lo + rbase, rpt), :])
            plsc.subcore_barrier()
            return carry

        lax.fori_loop(0, -(-n_chunks // _NCORES), pair_body, 0)

    return k(x_aug, src, dst)


_BN = 1024


def _tc_body(n_rel, rel_ids, *refs):
    out_ref = refs[-1]
    bias_ref = refs[-2]
    wc_ref = refs[-3]
    basis_ref = refs[-4]
    acc = None
    for i in range(n_rel):
        agg_ref = refs[2 * i]
        deg_ref = refs[2 * i + 1]
        r = rel_ids[i]
        w = (wc_ref[r, 0] * basis_ref[0]
             + wc_ref[r, 1] * basis_ref[1])
        inv = 1.0 / jnp.maximum(deg_ref[:, 0:1], 1.0)
        part = jnp.dot(agg_ref[:] * inv, w,
                       preferred_element_type=jnp.float32)
        acc = part if acc is None else acc + part
    out_ref[:] = acc + bias_ref[:]


def _norm_matmul(aggdegs, rel_ids, basis, w_comp, bias):
    n_rel = len(aggdegs)
    n_pad = aggdegs[0][0].shape[0]
    grid = (n_pad // _BN,)
    in_specs = []
    args = []
    for agg, deg in aggdegs:
        in_specs.append(pl.BlockSpec((_BN, 128), lambda i: (i, 0)))
        in_specs.append(pl.BlockSpec((_BN, _L), lambda i: (i, 0)))
        args += [agg, deg]
    in_specs.append(pl.BlockSpec((2, 128, 128), lambda i: (0, 0, 0)))
    in_specs.append(pl.BlockSpec(memory_space=pltpu.SMEM))
    in_specs.append(pl.BlockSpec((1, 128), lambda i: (0, 0)))
    args += [basis, w_comp, bias.reshape(1, 128)]
    return pl.pallas_call(
        functools.partial(_tc_body, n_rel, rel_ids),
        grid=grid,
        in_specs=in_specs,
        out_specs=pl.BlockSpec((_BN, 128), lambda i: (i, 0)),
        out_shape=jax.ShapeDtypeStruct((n_pad, 128), jnp.float32),
    )(*args)


def kernel(x_author, x_field_of_study, x_institution, x_paper,
           writes_src, writes_dst, rev_writes_src, rev_writes_dst,
           cites_src, cites_dst, aff_src, aff_dst, topic_src, topic_dst,
           basis, w_comp, bias):
    n_author = x_author.shape[0]
    n_paper = x_paper.shape[0]
    n_inst = x_institution.shape[0]
    n_fos = x_field_of_study.shape[0]

    xa_aug = _augment(x_author)
    xp_aug = _augment(x_paper)

    agg_w, deg_w = _seg_aggregate(xa_aug, writes_src, writes_dst, n_paper)
    agg_c, deg_c = _seg_aggregate(xp_aug, cites_src, cites_dst, n_paper)
    agg_r, deg_r = _seg_aggregate(xp_aug, rev_writes_src, rev_writes_dst,
                                  n_author)
    agg_a, deg_a = _seg_aggregate(xa_aug, aff_src, aff_dst, n_inst)
    agg_t, deg_t = _seg_aggregate(xp_aug, topic_src, topic_dst, n_fos)

    out_paper = _norm_matmul([(agg_w, deg_w), (agg_c, deg_c)], [0, 2],
                             basis, w_comp, bias)[:n_paper]
    out_author = _norm_matmul([(agg_r, deg_r)], [1],
                              basis, w_comp, bias)[:n_author]
    out_inst = _norm_matmul([(agg_a, deg_a)], [3],
                            basis, w_comp, bias)[:n_inst]
    out_fos = _norm_matmul([(agg_t, deg_t)], [4],
                           basis, w_comp, bias)[:n_fos]
    return (out_author, out_fos, out_inst, out_paper)


# speedup vs baseline: 55.0002x; 55.0002x over previous
import functools

import jax
import jax.numpy as jnp
from jax import lax
from jax.experimental import pallas as pl
from jax.experimental.pallas import tpu as pltpu
from jax.experimental.pallas import tpu_sc as plsc

_L = 16
_NTILES = 16
_NCORES = 2
_CHUNK = 9216
_BATCH = 128
_SBLK = 2048
_ZR = 32
_W = 144


def _ceil_to(x, m):
    return (x + m - 1) // m * m


def _augment(x):
    n = x.shape[0]
    return jnp.concatenate(
        [x, jnp.ones((n, 1), jnp.float32),
         jnp.zeros((n, _W - 129), jnp.float32)], axis=1)


def _seg_aggregate(x_aug, src, dst, n_dst):
    e = src.shape[0]
    e_pad = _ceil_to(e, _NTILES * _SBLK)
    if e_pad != e:
        pad = e_pad - e
        src = jnp.concatenate([src, jnp.zeros((pad,), jnp.int32)])
        dst = jnp.concatenate([dst, jnp.full((pad,), jnp.int32(1 << 30))])
    epw = e_pad // _NTILES
    n_sblocks = epw // _SBLK
    n_chunks = -(-n_dst // _CHUNK)
    n_pad = n_chunks * _CHUNK
    rpt = _CHUNK // _NTILES

    mesh = plsc.VectorSubcoreMesh(core_axis_name="c", subcore_axis_name="s")

    @functools.partial(
        pl.kernel,
        out_type=(
            jax.ShapeDtypeStruct((n_pad, 128), jnp.float32),
            jax.ShapeDtypeStruct((n_pad, _L), jnp.float32),
        ),
        mesh=mesh,
        compiler_params=pltpu.CompilerParams(
            use_tc_tiling_on_sc=False, needs_layout_passes=False),
        scratch_types=dict(
            esrc=pltpu.VMEM((_SBLK,), jnp.int32),
            edst=pltpu.VMEM((_SBLK,), jnp.int32),
            csrc=pltpu.VMEM((_BATCH + 2 * _L,), jnp.int32),
            cdst=pltpu.VMEM((_BATCH + 2 * _L,), jnp.int32),
            rows=pltpu.VMEM((_BATCH, _W), jnp.float32),
            gidx=pltpu.VMEM((_BATCH,), jnp.int32),
            sidx=pltpu.VMEM((_BATCH,), jnp.int32),
            zbuf=pltpu.VMEM((_ZR, _W), jnp.float32),
            acc_sh=pltpu.VMEM_SHARED((_CHUNK + _L, _W), jnp.float32),
        ),
    )
    def k(x_hbm, src_hbm, dst_hbm, agg_hbm, deg_hbm, *,
          esrc, edst, csrc, cdst, rows, gidx, sidx, zbuf, acc_sh):
        cid = lax.axis_index("c")
        sid = lax.axis_index("s")
        ebase = sid * epw

        for i in range(_ZR):
            for j in range(_W // _L):
                zbuf[i, pl.ds(j * _L, _L)] = jnp.zeros((_L,), jnp.float32)

        def fire_batch():
            for j in range(_BATCH // _L):
                gidx[pl.ds(j * _L, _L)] = csrc[pl.ds(j * _L, _L)]
                sidx[pl.ds(j * _L, _L)] = cdst[pl.ds(j * _L, _L)]
            pltpu.sync_copy(x_hbm.at[gidx], rows)
            pltpu.sync_copy(rows, acc_sh.at[sidx], add=True)

        def scan_vec(lo, off, cnt):
            s_v = esrc[pl.ds(off, _L)]
            d_v = edst[pl.ds(off, _L)]
            m = (d_v >= lo) & (d_v < lo + _CHUNK)
            mi = m.astype(jnp.int32)
            pos = cnt + plsc.cumsum(mi) - mi
            plsc.store_scatter(csrc, [pos], s_v, mask=m)
            plsc.store_scatter(cdst, [pos], d_v - lo, mask=m)
            cnt = cnt + plsc.all_reduce_population_count(m)

            def flush():
                fire_batch()
                spill_s = csrc[pl.ds(_BATCH, _L)]
                spill_d = cdst[pl.ds(_BATCH, _L)]
                csrc[pl.ds(0, _L)] = spill_s
                cdst[pl.ds(0, _L)] = spill_d
                return cnt - _BATCH

            return lax.cond(jnp.all(cnt >= _BATCH), flush, lambda: cnt)

        def final_flush(cnt):
            lanes = lax.iota(jnp.int32, _L)
            for j in range(_BATCH // _L):
                keep = (lanes + (j * _L)) < cnt
                sj = csrc[pl.ds(j * _L, _L)]
                dj = cdst[pl.ds(j * _L, _L)]
                csrc[pl.ds(j * _L, _L)] = jnp.where(keep, sj, 0)
                cdst[pl.ds(j * _L, _L)] = jnp.where(keep, dj, _CHUNK)
            fire_batch()

        rbase = sid * rpt
        def pair_body(i, carry):
            my_chunk = i * _NCORES + cid
            active = my_chunk < n_chunks
            lo = my_chunk * _CHUNK

            @pl.when(active)
            def _zero():
                def zloop(z, c2):
                    pltpu.sync_copy(
                        zbuf, acc_sh.at[pl.ds(rbase + z * _ZR, _ZR), :])
                    return c2
                lax.fori_loop(0, rpt // _ZR, zloop, 0)
            plsc.subcore_barrier()

            @pl.when(active)
            def _scan():
                def sblock(sb, cnt):
                    pltpu.sync_copy(
                        src_hbm.at[pl.ds(ebase + sb * _SBLK, _SBLK)], esrc)
                    pltpu.sync_copy(
                        dst_hbm.at[pl.ds(ebase + sb * _SBLK, _SBLK)], edst)
                    def vec(v, c2):
                        return scan_vec(lo, v * _L, c2)
                    return lax.fori_loop(0, _SBLK // _L, vec, cnt)
                cnt = lax.fori_loop(0, n_sblocks, sblock,
                                    jnp.zeros((_L,), jnp.int32))
                final_flush(cnt)
            plsc.subcore_barrier()

            @pl.when(active)
            def _writeout():
                pltpu.sync_copy(
                    acc_sh.at[pl.ds(rbase, rpt), pl.ds(0, 128)],
                    agg_hbm.at[pl.ds(lo + rbase, rpt), :])
                pltpu.sync_copy(
                    acc_sh.at[pl.ds(rbase, rpt), pl.ds(128, _L)],
                    deg_hbm.at[pl.ds(lo + rbase, rpt), :])
            plsc.subcore_barrier()
            return carry

        lax.fori_loop(0, -(-n_chunks // _NCORES), pair_body, 0)

    return k(x_aug, src, dst)


_BN = 1024


def _tc_body(n_rel, rel_ids, *refs):
    out_ref = refs[-1]
    bias_ref = refs[-2]
    wc_ref = refs[-3]
    basis_ref = refs[-4]
    acc = None
    for i in range(n_rel):
        agg_ref = refs[2 * i]
        deg_ref = refs[2 * i + 1]
        r = rel_ids[i]
        w = (wc_ref[r, 0] * basis_ref[0]
             + wc_ref[r, 1] * basis_ref[1])
        inv = 1.0 / jnp.maximum(deg_ref[:, 0:1], 1.0)
        part = jnp.dot(agg_ref[:] * inv, w,
                       preferred_element_type=jnp.float32)
        acc = part if acc is None else acc + part
    out_ref[:] = acc + bias_ref[:]


def _norm_matmul(aggdegs, rel_ids, basis, w_comp, bias):
    n_rel = len(aggdegs)
    n_pad = aggdegs[0][0].shape[0]
    grid = (n_pad // _BN,)
    in_specs = []
    args = []
    for agg, deg in aggdegs:
        in_specs.append(pl.BlockSpec((_BN, 128), lambda i: (i, 0)))
        in_specs.append(pl.BlockSpec((_BN, _L), lambda i: (i, 0)))
        args += [agg, deg]
    in_specs.append(pl.BlockSpec((2, 128, 128), lambda i: (0, 0, 0)))
    in_specs.append(pl.BlockSpec(memory_space=pltpu.SMEM))
    in_specs.append(pl.BlockSpec((1, 128), lambda i: (0, 0)))
    args += [basis, w_comp, bias.reshape(1, 128)]
    return pl.pallas_call(
        functools.partial(_tc_body, n_rel, rel_ids),
        grid=grid,
        in_specs=in_specs,
        out_specs=pl.BlockSpec((_BN, 128), lambda i: (i, 0)),
        out_shape=jax.ShapeDtypeStruct((n_pad, 128), jnp.float32),
    )(*args)


def kernel(x_author, x_field_of_study, x_institution, x_paper,
           writes_src, writes_dst, rev_writes_src, rev_writes_dst,
           cites_src, cites_dst, aff_src, aff_dst, topic_src, topic_dst,
           basis, w_comp, bias):
    n_author = x_author.shape[0]
    n_paper = x_paper.shape[0]
    n_inst = x_institution.shape[0]
    n_fos = x_field_of_study.shape[0]

    xa_aug = _augment(x_author)
    xp_aug = _augment(x_paper)

    agg_w, deg_w = _seg_aggregate(xa_aug, writes_src, writes_dst, n_paper)
    agg_c, deg_c = _seg_aggregate(xp_aug, cites_src, cites_dst, n_paper)
    agg_r, deg_r = _seg_aggregate(xp_aug, rev_writes_src, rev_writes_dst,
                                  n_author)
    agg_a, deg_a = _seg_aggregate(xa_aug, aff_src, aff_dst, n_inst)
    agg_t, deg_t = _seg_aggregate(xp_aug, topic_src, topic_dst, n_fos)

    out_paper = _norm_matmul([(agg_w, deg_w), (agg_c, deg_c)], [0, 2],
                             basis, w_comp, bias)[:n_paper]
    out_author = _norm_matmul([(agg_r, deg_r)], [1],
                              basis, w_comp, bias)[:n_author]
    out_inst = _norm_matmul([(agg_a, deg_a)], [3],
                            basis, w_comp, bias)[:n_inst]
    out_fos = _norm_matmul([(agg_t, deg_t)], [4],
                           basis, w_comp, bias)[:n_fos]
    return (out_author, out_fos, out_inst, out_paper)


# speedup vs baseline: 55.0623x; 1.0011x over previous
import functools

import jax
import jax.numpy as jnp
from jax import lax
from jax.experimental import pallas as pl
from jax.experimental.pallas import tpu as pltpu
from jax.experimental.pallas import tpu_sc as plsc

_L = 16
_NTILES = 16
_NCORES = 2
_CHUNK = 9216
_BATCH = 128
_SBLK = 2048
_ZR = 96
_W = 144


def _ceil_to(x, m):
    return (x + m - 1) // m * m


def _augment(x):
    n = x.shape[0]
    return jnp.concatenate(
        [x, jnp.ones((n, 1), jnp.float32),
         jnp.zeros((n, _W - 129), jnp.float32)], axis=1)


def _seg_aggregate(x_aug, src, dst, n_dst):
    e = src.shape[0]
    e_pad = _ceil_to(e, _NTILES * _SBLK)
    if e_pad != e:
        pad = e_pad - e
        src = jnp.concatenate([src, jnp.zeros((pad,), jnp.int32)])
        dst = jnp.concatenate([dst, jnp.full((pad,), jnp.int32(1 << 30))])
    epw = e_pad // _NTILES
    n_sblocks = epw // _SBLK
    n_chunks = -(-n_dst // _CHUNK)
    n_pad = n_chunks * _CHUNK
    rpt = _CHUNK // _NTILES

    mesh = plsc.VectorSubcoreMesh(core_axis_name="c", subcore_axis_name="s")

    @functools.partial(
        pl.kernel,
        out_type=(
            jax.ShapeDtypeStruct((n_pad, 128), jnp.float32),
            jax.ShapeDtypeStruct((n_pad, _L), jnp.float32),
        ),
        mesh=mesh,
        compiler_params=pltpu.CompilerParams(
            use_tc_tiling_on_sc=False, needs_layout_passes=False),
        scratch_types=dict(
            esrc=pltpu.VMEM((_SBLK,), jnp.int32),
            edst=pltpu.VMEM((_SBLK,), jnp.int32),
            csrc=pltpu.VMEM((_BATCH + 2 * _L,), jnp.int32),
            cdst=pltpu.VMEM((_BATCH + 2 * _L,), jnp.int32),
            rows=pltpu.VMEM((_BATCH, _W), jnp.float32),
            gidx=pltpu.VMEM((_BATCH,), jnp.int32),
            sidx=pltpu.VMEM((_BATCH,), jnp.int32),
            zbuf=pltpu.VMEM((_ZR, _W), jnp.float32),
            acc_sh=pltpu.VMEM_SHARED((_CHUNK + _L, _W), jnp.float32),
        ),
    )
    def k(x_hbm, src_hbm, dst_hbm, agg_hbm, deg_hbm, *,
          esrc, edst, csrc, cdst, rows, gidx, sidx, zbuf, acc_sh):
        cid = lax.axis_index("c")
        sid = lax.axis_index("s")
        ebase = sid * epw

        for i in range(_ZR):
            for j in range(_W // _L):
                zbuf[i, pl.ds(j * _L, _L)] = jnp.zeros((_L,), jnp.float32)

        def fire_batch():
            for j in range(_BATCH // _L):
                gidx[pl.ds(j * _L, _L)] = csrc[pl.ds(j * _L, _L)]
                sidx[pl.ds(j * _L, _L)] = cdst[pl.ds(j * _L, _L)]
            pltpu.sync_copy(x_hbm.at[gidx], rows)
            pltpu.sync_copy(rows, acc_sh.at[sidx], add=True)

        def scan_vec(lo, off, cnt):
            s_v = esrc[pl.ds(off, _L)]
            d_v = edst[pl.ds(off, _L)]
            m = (d_v >= lo) & (d_v < lo + _CHUNK)
            mi = m.astype(jnp.int32)
            pos = cnt + plsc.cumsum(mi) - mi
            plsc.store_scatter(csrc, [pos], s_v, mask=m)
            plsc.store_scatter(cdst, [pos], d_v - lo, mask=m)
            cnt = cnt + plsc.all_reduce_population_count(m)

            def flush():
                fire_batch()
                spill_s = csrc[pl.ds(_BATCH, _L)]
                spill_d = cdst[pl.ds(_BATCH, _L)]
                csrc[pl.ds(0, _L)] = spill_s
                cdst[pl.ds(0, _L)] = spill_d
                return cnt - _BATCH

            return lax.cond(jnp.all(cnt >= _BATCH), flush, lambda: cnt)

        def final_flush(cnt):
            lanes = lax.iota(jnp.int32, _L)
            for j in range(_BATCH // _L):
                keep = (lanes + (j * _L)) < cnt
                sj = csrc[pl.ds(j * _L, _L)]
                dj = cdst[pl.ds(j * _L, _L)]
                csrc[pl.ds(j * _L, _L)] = jnp.where(keep, sj, 0)
                cdst[pl.ds(j * _L, _L)] = jnp.where(keep, dj, _CHUNK)
            fire_batch()

        rbase = sid * rpt
        def pair_body(i, carry):
            my_chunk = i * _NCORES + cid
            active = my_chunk < n_chunks
            lo = my_chunk * _CHUNK

            @pl.when(active)
            def _zero():
                def zloop(z, c2):
                    pltpu.sync_copy(
                        zbuf, acc_sh.at[pl.ds(rbase + z * _ZR, _ZR), :])
                    return c2
                lax.fori_loop(0, rpt // _ZR, zloop, 0)
            plsc.subcore_barrier()

            @pl.when(active)
            def _scan():
                def sblock(sb, cnt):
                    pltpu.sync_copy(
                        src_hbm.at[pl.ds(ebase + sb * _SBLK, _SBLK)], esrc)
                    pltpu.sync_copy(
                        dst_hbm.at[pl.ds(ebase + sb * _SBLK, _SBLK)], edst)
                    def vec(v, c2):
                        return scan_vec(lo, v * _L, c2)
                    return lax.fori_loop(0, _SBLK // _L, vec, cnt)
                cnt = lax.fori_loop(0, n_sblocks, sblock,
                                    jnp.zeros((_L,), jnp.int32))
                final_flush(cnt)
            plsc.subcore_barrier()

            @pl.when(active)
            def _writeout():
                pltpu.sync_copy(
                    acc_sh.at[pl.ds(rbase, rpt), pl.ds(0, 128)],
                    agg_hbm.at[pl.ds(lo + rbase, rpt), :])
                pltpu.sync_copy(
                    acc_sh.at[pl.ds(rbase, rpt), pl.ds(128, _L)],
                    deg_hbm.at[pl.ds(lo + rbase, rpt), :])
            plsc.subcore_barrier()
            return carry

        lax.fori_loop(0, -(-n_chunks // _NCORES), pair_body, 0)

    return k(x_aug, src, dst)


_BN = 1024


def _tc_body(n_rel, rel_ids, *refs):
    out_ref = refs[-1]
    bias_ref = refs[-2]
    wc_ref = refs[-3]
    basis_ref = refs[-4]
    acc = None
    for i in range(n_rel):
        agg_ref = refs[2 * i]
        deg_ref = refs[2 * i + 1]
        r = rel_ids[i]
        w = (wc_ref[r, 0] * basis_ref[0]
             + wc_ref[r, 1] * basis_ref[1])
        inv = 1.0 / jnp.maximum(deg_ref[:, 0:1], 1.0)
        part = jnp.dot(agg_ref[:] * inv, w,
                       preferred_element_type=jnp.float32)
        acc = part if acc is None else acc + part
    out_ref[:] = acc + bias_ref[:]


def _norm_matmul(aggdegs, rel_ids, basis, w_comp, bias):
    n_rel = len(aggdegs)
    n_pad = aggdegs[0][0].shape[0]
    grid = (n_pad // _BN,)
    in_specs = []
    args = []
    for agg, deg in aggdegs:
        in_specs.append(pl.BlockSpec((_BN, 128), lambda i: (i, 0)))
        in_specs.append(pl.BlockSpec((_BN, _L), lambda i: (i, 0)))
        args += [agg, deg]
    in_specs.append(pl.BlockSpec((2, 128, 128), lambda i: (0, 0, 0)))
    in_specs.append(pl.BlockSpec(memory_space=pltpu.SMEM))
    in_specs.append(pl.BlockSpec((1, 128), lambda i: (0, 0)))
    args += [basis, w_comp, bias.reshape(1, 128)]
    return pl.pallas_call(
        functools.partial(_tc_body, n_rel, rel_ids),
        grid=grid,
        in_specs=in_specs,
        out_specs=pl.BlockSpec((_BN, 128), lambda i: (i, 0)),
        out_shape=jax.ShapeDtypeStruct((n_pad, 128), jnp.float32),
    )(*args)


def kernel(x_author, x_field_of_study, x_institution, x_paper,
           writes_src, writes_dst, rev_writes_src, rev_writes_dst,
           cites_src, cites_dst, aff_src, aff_dst, topic_src, topic_dst,
           basis, w_comp, bias):
    n_author = x_author.shape[0]
    n_paper = x_paper.shape[0]
    n_inst = x_institution.shape[0]
    n_fos = x_field_of_study.shape[0]

    xa_aug = _augment(x_author)
    xp_aug = _augment(x_paper)

    agg_w, deg_w = _seg_aggregate(xa_aug, writes_src, writes_dst, n_paper)
    agg_c, deg_c = _seg_aggregate(xp_aug, cites_src, cites_dst, n_paper)
    agg_r, deg_r = _seg_aggregate(xp_aug, rev_writes_src, rev_writes_dst,
                                  n_author)
    agg_a, deg_a = _seg_aggregate(xa_aug, aff_src, aff_dst, n_inst)
    agg_t, deg_t = _seg_aggregate(xp_aug, topic_src, topic_dst, n_fos)

    out_paper = _norm_matmul([(agg_w, deg_w), (agg_c, deg_c)], [0, 2],
                             basis, w_comp, bias)[:n_paper]
    out_author = _norm_matmul([(agg_r, deg_r)], [1],
                              basis, w_comp, bias)[:n_author]
    out_inst = _norm_matmul([(agg_a, deg_a)], [3],
                            basis, w_comp, bias)[:n_inst]
    out_fos = _norm_matmul([(agg_t, deg_t)], [4],
                           basis, w_comp, bias)[:n_fos]
    return (out_author, out_fos, out_inst, out_paper)


# speedup vs baseline: 59.3393x; 1.0777x over previous
import functools

import jax
import jax.numpy as jnp
from jax import lax
from jax.experimental import pallas as pl
from jax.experimental.pallas import tpu as pltpu
from jax.experimental.pallas import tpu_sc as plsc

_L = 16
_NTILES = 16
_NCORES = 2
_CHUNK = 10240
_BATCH = 128
_SBLK = 2048
_ZR = 32
_W = 144


def _ceil_to(x, m):
    return (x + m - 1) // m * m


def _augment(x):
    n = x.shape[0]
    return jnp.concatenate(
        [x, jnp.ones((n, 1), jnp.float32),
         jnp.zeros((n, _W - 129), jnp.float32)], axis=1)


def _seg_aggregate(x_aug, src, dst, n_dst):
    e = src.shape[0]
    e_pad = _ceil_to(e, _NTILES * _SBLK)
    if e_pad != e:
        pad = e_pad - e
        src = jnp.concatenate([src, jnp.zeros((pad,), jnp.int32)])
        dst = jnp.concatenate([dst, jnp.full((pad,), jnp.int32(1 << 30))])
    epw = e_pad // _NTILES
    n_sblocks = epw // _SBLK
    n_chunks = -(-n_dst // _CHUNK)
    n_pad = n_chunks * _CHUNK
    rpt = _CHUNK // _NTILES

    mesh = plsc.VectorSubcoreMesh(core_axis_name="c", subcore_axis_name="s")

    @functools.partial(
        pl.kernel,
        out_type=(
            jax.ShapeDtypeStruct((n_pad, 128), jnp.float32),
            jax.ShapeDtypeStruct((n_pad, _L), jnp.float32),
        ),
        mesh=mesh,
        compiler_params=pltpu.CompilerParams(
            use_tc_tiling_on_sc=False, needs_layout_passes=False),
        scratch_types=dict(
            esrc=pltpu.VMEM((_SBLK,), jnp.int32),
            edst=pltpu.VMEM((_SBLK,), jnp.int32),
            csrc=pltpu.VMEM((_BATCH + 2 * _L,), jnp.int32),
            cdst=pltpu.VMEM((_BATCH + 2 * _L,), jnp.int32),
            rows=pltpu.VMEM((_BATCH, _W), jnp.float32),
            gidx=pltpu.VMEM((_BATCH,), jnp.int32),
            sidx=pltpu.VMEM((_BATCH,), jnp.int32),
            zbuf=pltpu.VMEM((_ZR, _W), jnp.float32),
            acc_sh=pltpu.VMEM_SHARED((_CHUNK + _L, _W), jnp.float32),
        ),
    )
    def k(x_hbm, src_hbm, dst_hbm, agg_hbm, deg_hbm, *,
          esrc, edst, csrc, cdst, rows, gidx, sidx, zbuf, acc_sh):
        cid = lax.axis_index("c")
        sid = lax.axis_index("s")
        ebase = sid * epw

        for i in range(_ZR):
            for j in range(_W // _L):
                zbuf[i, pl.ds(j * _L, _L)] = jnp.zeros((_L,), jnp.float32)

        def fire_batch():
            for j in range(_BATCH // _L):
                gidx[pl.ds(j * _L, _L)] = csrc[pl.ds(j * _L, _L)]
                sidx[pl.ds(j * _L, _L)] = cdst[pl.ds(j * _L, _L)]
            pltpu.sync_copy(x_hbm.at[gidx], rows)
            pltpu.sync_copy(rows, acc_sh.at[sidx], add=True)

        def scan_vec(lo, off, cnt):
            s_v = esrc[pl.ds(off, _L)]
            d_v = edst[pl.ds(off, _L)]
            m = (d_v >= lo) & (d_v < lo + _CHUNK)
            mi = m.astype(jnp.int32)
            pos = cnt + plsc.cumsum(mi) - mi
            plsc.store_scatter(csrc, [pos], s_v, mask=m)
            plsc.store_scatter(cdst, [pos], d_v - lo, mask=m)
            cnt = cnt + plsc.all_reduce_population_count(m)

            def flush():
                fire_batch()
                spill_s = csrc[pl.ds(_BATCH, _L)]
                spill_d = cdst[pl.ds(_BATCH, _L)]
                csrc[pl.ds(0, _L)] = spill_s
                cdst[pl.ds(0, _L)] = spill_d
                return cnt - _BATCH

            return lax.cond(jnp.all(cnt >= _BATCH), flush, lambda: cnt)

        def final_flush(cnt):
            lanes = lax.iota(jnp.int32, _L)
            for j in range(_BATCH // _L):
                keep = (lanes + (j * _L)) < cnt
                sj = csrc[pl.ds(j * _L, _L)]
                dj = cdst[pl.ds(j * _L, _L)]
                csrc[pl.ds(j * _L, _L)] = jnp.where(keep, sj, 0)
                cdst[pl.ds(j * _L, _L)] = jnp.where(keep, dj, _CHUNK)
            fire_batch()

        rbase = sid * rpt
        def pair_body(i, carry):
            my_chunk = i * _NCORES + cid
            active = my_chunk < n_chunks
            lo = my_chunk * _CHUNK

            @pl.when(active)
            def _zero():
                def zloop(z, c2):
                    pltpu.sync_copy(
                        zbuf, acc_sh.at[pl.ds(rbase + z * _ZR, _ZR), :])
                    return c2
                lax.fori_loop(0, rpt // _ZR, zloop, 0)
            plsc.subcore_barrier()

            @pl.when(active)
            def _scan():
                def sblock(sb, cnt):
                    pltpu.sync_copy(
                        src_hbm.at[pl.ds(ebase + sb * _SBLK, _SBLK)], esrc)
                    pltpu.sync_copy(
                        dst_hbm.at[pl.ds(ebase + sb * _SBLK, _SBLK)], edst)
                    def vec(v, c2):
                        return scan_vec(lo, v * _L, c2)
                    return lax.fori_loop(0, _SBLK // _L, vec, cnt)
                cnt = lax.fori_loop(0, n_sblocks, sblock,
                                    jnp.zeros((_L,), jnp.int32))
                final_flush(cnt)
            plsc.subcore_barrier()

            @pl.when(active)
            def _writeout():
                pltpu.sync_copy(
                    acc_sh.at[pl.ds(rbase, rpt), pl.ds(0, 128)],
                    agg_hbm.at[pl.ds(lo + rbase, rpt), :])
                pltpu.sync_copy(
                    acc_sh.at[pl.ds(rbase, rpt), pl.ds(128, _L)],
                    deg_hbm.at[pl.ds(lo + rbase, rpt), :])
            plsc.subcore_barrier()
            return carry

        lax.fori_loop(0, -(-n_chunks // _NCORES), pair_body, 0)

    return k(x_aug, src, dst)


_BN = 1024


def _tc_body(n_rel, rel_ids, *refs):
    out_ref = refs[-1]
    bias_ref = refs[-2]
    wc_ref = refs[-3]
    basis_ref = refs[-4]
    acc = None
    for i in range(n_rel):
        agg_ref = refs[2 * i]
        deg_ref = refs[2 * i + 1]
        r = rel_ids[i]
        w = (wc_ref[r, 0] * basis_ref[0]
             + wc_ref[r, 1] * basis_ref[1])
        inv = 1.0 / jnp.maximum(deg_ref[:, 0:1], 1.0)
        part = jnp.dot(agg_ref[:] * inv, w,
                       preferred_element_type=jnp.float32)
        acc = part if acc is None else acc + part
    out_ref[:] = acc + bias_ref[:]


def _norm_matmul(aggdegs, rel_ids, basis, w_comp, bias):
    n_rel = len(aggdegs)
    n_pad = aggdegs[0][0].shape[0]
    grid = (n_pad // _BN,)
    in_specs = []
    args = []
    for agg, deg in aggdegs:
        in_specs.append(pl.BlockSpec((_BN, 128), lambda i: (i, 0)))
        in_specs.append(pl.BlockSpec((_BN, _L), lambda i: (i, 0)))
        args += [agg, deg]
    in_specs.append(pl.BlockSpec((2, 128, 128), lambda i: (0, 0, 0)))
    in_specs.append(pl.BlockSpec(memory_space=pltpu.SMEM))
    in_specs.append(pl.BlockSpec((1, 128), lambda i: (0, 0)))
    args += [basis, w_comp, bias.reshape(1, 128)]
    return pl.pallas_call(
        functools.partial(_tc_body, n_rel, rel_ids),
        grid=grid,
        in_specs=in_specs,
        out_specs=pl.BlockSpec((_BN, 128), lambda i: (i, 0)),
        out_shape=jax.ShapeDtypeStruct((n_pad, 128), jnp.float32),
    )(*args)


def kernel(x_author, x_field_of_study, x_institution, x_paper,
           writes_src, writes_dst, rev_writes_src, rev_writes_dst,
           cites_src, cites_dst, aff_src, aff_dst, topic_src, topic_dst,
           basis, w_comp, bias):
    n_author = x_author.shape[0]
    n_paper = x_paper.shape[0]
    n_inst = x_institution.shape[0]
    n_fos = x_field_of_study.shape[0]

    xa_aug = _augment(x_author)
    xp_aug = _augment(x_paper)

    agg_w, deg_w = _seg_aggregate(xa_aug, writes_src, writes_dst, n_paper)
    agg_c, deg_c = _seg_aggregate(xp_aug, cites_src, cites_dst, n_paper)
    agg_r, deg_r = _seg_aggregate(xp_aug, rev_writes_src, rev_writes_dst,
                                  n_author)
    agg_a, deg_a = _seg_aggregate(xa_aug, aff_src, aff_dst, n_inst)
    agg_t, deg_t = _seg_aggregate(xp_aug, topic_src, topic_dst, n_fos)

    out_paper = _norm_matmul([(agg_w, deg_w), (agg_c, deg_c)], [0, 2],
                             basis, w_comp, bias)[:n_paper]
    out_author = _norm_matmul([(agg_r, deg_r)], [1],
                              basis, w_comp, bias)[:n_author]
    out_inst = _norm_matmul([(agg_a, deg_a)], [3],
                            basis, w_comp, bias)[:n_inst]
    out_fos = _norm_matmul([(agg_t, deg_t)], [4],
                           basis, w_comp, bias)[:n_fos]
    return (out_author, out_fos, out_inst, out_paper)


# speedup vs baseline: 59.5930x; 1.0043x over previous
import functools

import jax
import jax.numpy as jnp
from jax import lax
from jax.experimental import pallas as pl
from jax.experimental.pallas import tpu as pltpu
from jax.experimental.pallas import tpu_sc as plsc

_L = 16
_NTILES = 16
_NCORES = 2
_CHUNK = 11264
_BATCH = 128
_SBLK = 2048
_ZR = 32
_W = 144


def _ceil_to(x, m):
    return (x + m - 1) // m * m


def _augment(x):
    n = x.shape[0]
    return jnp.concatenate(
        [x, jnp.ones((n, 1), jnp.float32),
         jnp.zeros((n, _W - 129), jnp.float32)], axis=1)


def _seg_aggregate(x_aug, src, dst, n_dst):
    e = src.shape[0]
    e_pad = _ceil_to(e, _NTILES * _SBLK)
    if e_pad != e:
        pad = e_pad - e
        src = jnp.concatenate([src, jnp.zeros((pad,), jnp.int32)])
        dst = jnp.concatenate([dst, jnp.full((pad,), jnp.int32(1 << 30))])
    epw = e_pad // _NTILES
    n_sblocks = epw // _SBLK
    n_chunks = -(-n_dst // _CHUNK)
    n_pad = n_chunks * _CHUNK
    rpt = _CHUNK // _NTILES

    mesh = plsc.VectorSubcoreMesh(core_axis_name="c", subcore_axis_name="s")

    @functools.partial(
        pl.kernel,
        out_type=(
            jax.ShapeDtypeStruct((n_pad, 128), jnp.float32),
            jax.ShapeDtypeStruct((n_pad, _L), jnp.float32),
        ),
        mesh=mesh,
        compiler_params=pltpu.CompilerParams(
            use_tc_tiling_on_sc=False, needs_layout_passes=False),
        scratch_types=dict(
            esrc=pltpu.VMEM((_SBLK,), jnp.int32),
            edst=pltpu.VMEM((_SBLK,), jnp.int32),
            csrc=pltpu.VMEM((_BATCH + 2 * _L,), jnp.int32),
            cdst=pltpu.VMEM((_BATCH + 2 * _L,), jnp.int32),
            rows=pltpu.VMEM((_BATCH, _W), jnp.float32),
            gidx=pltpu.VMEM((_BATCH,), jnp.int32),
            sidx=pltpu.VMEM((_BATCH,), jnp.int32),
            zbuf=pltpu.VMEM((_ZR, _W), jnp.float32),
            acc_sh=pltpu.VMEM_SHARED((_CHUNK + _L, _W), jnp.float32),
        ),
    )
    def k(x_hbm, src_hbm, dst_hbm, agg_hbm, deg_hbm, *,
          esrc, edst, csrc, cdst, rows, gidx, sidx, zbuf, acc_sh):
        cid = lax.axis_index("c")
        sid = lax.axis_index("s")
        ebase = sid * epw

        for i in range(_ZR):
            for j in range(_W // _L):
                zbuf[i, pl.ds(j * _L, _L)] = jnp.zeros((_L,), jnp.float32)

        def fire_batch():
            for j in range(_BATCH // _L):
                gidx[pl.ds(j * _L, _L)] = csrc[pl.ds(j * _L, _L)]
                sidx[pl.ds(j * _L, _L)] = cdst[pl.ds(j * _L, _L)]
            pltpu.sync_copy(x_hbm.at[gidx], rows)
            pltpu.sync_copy(rows, acc_sh.at[sidx], add=True)

        def scan_vec(lo, off, cnt):
            s_v = esrc[pl.ds(off, _L)]
            d_v = edst[pl.ds(off, _L)]
            m = (d_v >= lo) & (d_v < lo + _CHUNK)
            mi = m.astype(jnp.int32)
            pos = cnt + plsc.cumsum(mi) - mi
            plsc.store_scatter(csrc, [pos], s_v, mask=m)
            plsc.store_scatter(cdst, [pos], d_v - lo, mask=m)
            cnt = cnt + plsc.all_reduce_population_count(m)

            def flush():
                fire_batch()
                spill_s = csrc[pl.ds(_BATCH, _L)]
                spill_d = cdst[pl.ds(_BATCH, _L)]
                csrc[pl.ds(0, _L)] = spill_s
                cdst[pl.ds(0, _L)] = spill_d
                return cnt - _BATCH

            return lax.cond(jnp.all(cnt >= _BATCH), flush, lambda: cnt)

        def final_flush(cnt):
            lanes = lax.iota(jnp.int32, _L)
            for j in range(_BATCH // _L):
                keep = (lanes + (j * _L)) < cnt
                sj = csrc[pl.ds(j * _L, _L)]
                dj = cdst[pl.ds(j * _L, _L)]
                csrc[pl.ds(j * _L, _L)] = jnp.where(keep, sj, 0)
                cdst[pl.ds(j * _L, _L)] = jnp.where(keep, dj, _CHUNK)
            fire_batch()

        rbase = sid * rpt
        def pair_body(i, carry):
            my_chunk = i * _NCORES + cid
            active = my_chunk < n_chunks
            lo = my_chunk * _CHUNK

            @pl.when(active)
            def _zero():
                def zloop(z, c2):
                    pltpu.sync_copy(
                        zbuf, acc_sh.at[pl.ds(rbase + z * _ZR, _ZR), :])
                    return c2
                lax.fori_loop(0, rpt // _ZR, zloop, 0)
            plsc.subcore_barrier()

            @pl.when(active)
            def _scan():
                def sblock(sb, cnt):
                    pltpu.sync_copy(
                        src_hbm.at[pl.ds(ebase + sb * _SBLK, _SBLK)], esrc)
                    pltpu.sync_copy(
                        dst_hbm.at[pl.ds(ebase + sb * _SBLK, _SBLK)], edst)
                    def vec(v, c2):
                        return scan_vec(lo, v * _L, c2)
                    return lax.fori_loop(0, _SBLK // _L, vec, cnt)
                cnt = lax.fori_loop(0, n_sblocks, sblock,
                                    jnp.zeros((_L,), jnp.int32))
                final_flush(cnt)
            plsc.subcore_barrier()

            @pl.when(active)
            def _writeout():
                pltpu.sync_copy(
                    acc_sh.at[pl.ds(rbase, rpt), pl.ds(0, 128)],
                    agg_hbm.at[pl.ds(lo + rbase, rpt), :])
                pltpu.sync_copy(
                    acc_sh.at[pl.ds(rbase, rpt), pl.ds(128, _L)],
                    deg_hbm.at[pl.ds(lo + rbase, rpt), :])
            plsc.subcore_barrier()
            return carry

        lax.fori_loop(0, -(-n_chunks // _NCORES), pair_body, 0)

    return k(x_aug, src, dst)


_BN = 1024


def _tc_body(n_rel, rel_ids, *refs):
    out_ref = refs[-1]
    bias_ref = refs[-2]
    wc_ref = refs[-3]
    basis_ref = refs[-4]
    acc = None
    for i in range(n_rel):
        agg_ref = refs[2 * i]
        deg_ref = refs[2 * i + 1]
        r = rel_ids[i]
        w = (wc_ref[r, 0] * basis_ref[0]
             + wc_ref[r, 1] * basis_ref[1])
        inv = 1.0 / jnp.maximum(deg_ref[:, 0:1], 1.0)
        part = jnp.dot(agg_ref[:] * inv, w,
                       preferred_element_type=jnp.float32)
        acc = part if acc is None else acc + part
    out_ref[:] = acc + bias_ref[:]


def _norm_matmul(aggdegs, rel_ids, basis, w_comp, bias):
    n_rel = len(aggdegs)
    n_pad = aggdegs[0][0].shape[0]
    grid = (n_pad // _BN,)
    in_specs = []
    args = []
    for agg, deg in aggdegs:
        in_specs.append(pl.BlockSpec((_BN, 128), lambda i: (i, 0)))
        in_specs.append(pl.BlockSpec((_BN, _L), lambda i: (i, 0)))
        args += [agg, deg]
    in_specs.append(pl.BlockSpec((2, 128, 128), lambda i: (0, 0, 0)))
    in_specs.append(pl.BlockSpec(memory_space=pltpu.SMEM))
    in_specs.append(pl.BlockSpec((1, 128), lambda i: (0, 0)))
    args += [basis, w_comp, bias.reshape(1, 128)]
    return pl.pallas_call(
        functools.partial(_tc_body, n_rel, rel_ids),
        grid=grid,
        in_specs=in_specs,
        out_specs=pl.BlockSpec((_BN, 128), lambda i: (i, 0)),
        out_shape=jax.ShapeDtypeStruct((n_pad, 128), jnp.float32),
    )(*args)


def kernel(x_author, x_field_of_study, x_institution, x_paper,
           writes_src, writes_dst, rev_writes_src, rev_writes_dst,
           cites_src, cites_dst, aff_src, aff_dst, topic_src, topic_dst,
           basis, w_comp, bias):
    n_author = x_author.shape[0]
    n_paper = x_paper.shape[0]
    n_inst = x_institution.shape[0]
    n_fos = x_field_of_study.shape[0]

    xa_aug = _augment(x_author)
    xp_aug = _augment(x_paper)

    agg_w, deg_w = _seg_aggregate(xa_aug, writes_src, writes_dst, n_paper)
    agg_c, deg_c = _seg_aggregate(xp_aug, cites_src, cites_dst, n_paper)
    agg_r, deg_r = _seg_aggregate(xp_aug, rev_writes_src, rev_writes_dst,
                                  n_author)
    agg_a, deg_a = _seg_aggregate(xa_aug, aff_src, aff_dst, n_inst)
    agg_t, deg_t = _seg_aggregate(xp_aug, topic_src, topic_dst, n_fos)

    out_paper = _norm_matmul([(agg_w, deg_w), (agg_c, deg_c)], [0, 2],
                             basis, w_comp, bias)[:n_paper]
    out_author = _norm_matmul([(agg_r, deg_r)], [1],
                              basis, w_comp, bias)[:n_author]
    out_inst = _norm_matmul([(agg_a, deg_a)], [3],
                            basis, w_comp, bias)[:n_inst]
    out_fos = _norm_matmul([(agg_t, deg_t)], [4],
                           basis, w_comp, bias)[:n_fos]
    return (out_author, out_fos, out_inst, out_paper)
